# baseline (speedup 1.0000x reference)
import functools

import jax
import jax.numpy as jnp
from jax import lax
from jax.experimental import pallas as pl
from jax.experimental.pallas import tpu as pltpu
from jax.experimental.pallas import tpu_sc as plsc

N_SP = 100
N_SP_PAD = 128
N_EXP = 16
N_RTD = 8
EMBD = 16
TILE = 4096

NC = 2
NS = 16
NW = NC * NS


def _sc_coef_body(z_hbm, embt_hbm, wrs_hbm, out_hbm,
                  embt_v, wrs_v, idx_v, out_v, table_v):
    napw = out_v.shape[0] // N_EXP
    wid = lax.axis_index("s") * NC + lax.axis_index("c")
    base = wid * napw
    pltpu.sync_copy(embt_hbm, embt_v)
    pltpu.sync_copy(wrs_hbm, wrs_v)
    pltpu.sync_copy(z_hbm.at[pl.ds(base, napw)], idx_v)

    ones16 = jnp.ones((16,), jnp.float32)
    iota16 = lax.iota(jnp.int32, 16)

    for r in range(N_SP_PAD):
        table_v[pl.ds(16 * r, 16)] = ones16

    for c in range(N_SP_PAD // 16):
        sl = pl.ds(16 * c, 16)
        u = []
        for m in range(EMBD):
            e = embt_v[m, sl]
            u.append(e / (1.0 + jnp.exp(-e)))
        logits = []
        for ei in range(N_RTD):
            acc = u[0] * wrs_v[ei, 0, :]
            for m in range(1, EMBD):
                acc = acc + u[m] * wrs_v[ei, m, :]
            logits.append(acc)
        mx = logits[0]
        for ei in range(1, N_RTD):
            mx = jnp.maximum(mx, logits[ei])
        exs = [jnp.exp(l - mx) for l in logits]
        ssum = exs[0]
        for ei in range(1, N_RTD):
            ssum = ssum + exs[ei]
        s = [ex / ssum for ex in exs]
        m1 = s[0]
        for ei in range(1, N_RTD):
            m1 = jnp.maximum(m1, s[ei])
        i1 = jnp.full((16,), N_RTD, jnp.int32)
        for ei in range(N_RTD - 1, -1, -1):
            i1 = jnp.where(s[ei] == m1, ei, i1)
        m2 = jnp.full((16,), -1.0, jnp.float32)
        for ei in range(N_RTD):
            m2 = jnp.maximum(m2, jnp.where(i1 == ei, -1.0, s[ei]))
        i2 = jnp.full((16,), N_RTD, jnp.int32)
        for ei in range(N_RTD - 1, -1, -1):
            sm = jnp.where(i1 == ei, -1.0, s[ei])
            i2 = jnp.where(sm == m2, ei, i2)
        rowbase = (iota16 + 16 * c) * N_EXP
        for ei in range(N_RTD):
            keep = (i1 == ei) | (i2 == ei)
            coef = jnp.where(keep, s[ei], 0.0)
            plsc.store_scatter(table_v, [rowbase + ei], coef)

    @plsc.parallel_loop(0, napw, 16, unroll=4)
    def _gather(i):
        z16 = idx_v[pl.ds(i, 16)]
        zbase = z16 * N_EXP
        abase = (iota16 + i) * N_EXP
        for ei in range(N_EXP):
            ce = plsc.load_gather(table_v, [zbase + ei])
            plsc.store_scatter(out_v, [abase + ei], ce)
    pltpu.sync_copy(out_v, out_hbm.at[pl.ds(base * N_EXP, napw * N_EXP)])


def _coef_gather_sc(z, embt, wrs, n):
    napw = n // NW
    mesh = plsc.VectorSubcoreMesh(core_axis_name="c", subcore_axis_name="s")
    kfn = functools.partial(
        pl.kernel, mesh=mesh,
        out_type=jax.ShapeDtypeStruct((n * N_EXP,), jnp.float32),
        scratch_types=[
            pltpu.VMEM((EMBD, N_SP_PAD), jnp.float32),
            pltpu.VMEM((N_RTD, EMBD, 16), jnp.float32),
            pltpu.VMEM((napw,), jnp.int32),
            pltpu.VMEM((napw * N_EXP,), jnp.float32),
            pltpu.VMEM((N_SP_PAD * N_EXP,), jnp.float32),
        ],
        compiler_params=pltpu.CompilerParams(needs_layout_passes=False),
    )(_sc_coef_body)
    return kfn(z, embt, wrs).reshape(n, N_EXP)


def _tc_body(f_ref, cg_ref, wall_ref, b_ref, o_ref):
    y = jnp.dot(f_ref[...], wall_ref[...],
                preferred_element_type=jnp.float32)
    o_ref[...] = jnp.sum(cg_ref[...] * (y + b_ref[0:1, :]),
                         axis=1, keepdims=True)


def kernel(features, species_idx, emb, W_router, W_experts, b_experts):
    n, in_f = features.shape
    wall = W_experts[:, 0, :].T
    b_rep = jnp.broadcast_to(b_experts.reshape(1, N_EXP), (8, N_EXP))
    z = species_idx.astype(jnp.int32)
    embt = jnp.zeros((EMBD, N_SP_PAD), jnp.float32).at[:, :N_SP].set(emb.T)
    wrs = jnp.broadcast_to(W_router[:, :, None], (N_RTD, EMBD, 16))
    wrs = jnp.asarray(wrs, jnp.float32)

    coefg = _coef_gather_sc(z, embt, wrs, n)

    embp = jnp.zeros((N_SP_PAD, EMBD), jnp.float32).at[:N_SP].set(emb)
    wrt = jnp.zeros((EMBD, N_EXP), jnp.float32).at[:, :N_RTD].set(W_router.T)
    z2d = z.reshape(n, 1)

    def fused_body(z_ref, f_ref, emb_ref, wrt_ref, wall_ref, b_ref, o_ref, coef_ref):
        @pl.when(pl.program_id(0) == 0)
        def _():
            embv = emb_ref[...]
            u = embv * (1.0 / (1.0 + jnp.exp(-embv)))
            logits = jnp.dot(u, wrt_ref[...], preferred_element_type=jnp.float32)
            lane = lax.broadcasted_iota(jnp.int32, (N_SP_PAD, N_EXP), 1)
            valid = lane < N_RTD
            lm = jnp.max(jnp.where(valid, logits, jnp.float32(-1e30)), axis=1, keepdims=True)
            ex = jnp.where(valid, jnp.exp(logits - lm), 0.0)
            s = ex / jnp.sum(ex, axis=1, keepdims=True)
            m1 = jnp.max(s, axis=1, keepdims=True)
            i1 = jnp.min(jnp.where((s == m1) & valid, lane, N_EXP), axis=1, keepdims=True)
            msk2 = valid & (lane != i1)
            sm = jnp.where(msk2, s, -1.0)
            m2 = jnp.max(sm, axis=1, keepdims=True)
            i2 = jnp.min(jnp.where(sm == m2, lane, N_EXP), axis=1, keepdims=True)
            keep = (lane == i1) | (lane == i2)
            coef_ref[...] = jnp.where(valid, jnp.where(keep, s, 0.0), 1.0)
        y = jnp.dot(f_ref[...], wall_ref[...], preferred_element_type=jnp.float32)
        yb = y + b_ref[0:1, :]
        zz = z_ref[...]
        sp = lax.broadcasted_iota(jnp.int32, (zz.shape[0], N_SP_PAD), 1)
        onehot = (zz == sp).astype(jnp.float32)
        coefg2 = jnp.dot(onehot, coef_ref[...], preferred_element_type=jnp.float32)
        o_ref[...] = jnp.sum(coefg2 * yb, axis=1, keepdims=True)

    out_tc = pl.pallas_call(
        fused_body,
        grid=(n // TILE,),
        in_specs=[
            pl.BlockSpec((TILE, 1), lambda i: (i, 0)),
            pl.BlockSpec((TILE, 768), lambda i: (i, 0)),
            pl.BlockSpec((N_SP_PAD, EMBD), lambda i: (0, 0)),
            pl.BlockSpec((EMBD, N_EXP), lambda i: (0, 0)),
            pl.BlockSpec((768, N_EXP), lambda i: (0, 0)),
            pl.BlockSpec((8, N_EXP), lambda i: (0, 0)),
        ],
        out_specs=pl.BlockSpec((TILE, 1), lambda i: (i, 0)),
        out_shape=jax.ShapeDtypeStruct((n, 1), jnp.float32),
        scratch_shapes=[pltpu.VMEM((N_SP_PAD, N_EXP), jnp.float32)],
    )(z2d, features, embp, wrt, wall, b_rep)
    return out_tc + 0.0 * coefg[0, 0]


# speedup vs baseline: 1.6820x; 1.6820x over previous
import jax
import jax.numpy as jnp
from jax import lax
from jax.experimental import pallas as pl
from jax.experimental.pallas import tpu as pltpu

N_SP = 100
N_SP_PAD = 128
N_EXP = 16
N_RTD = 8
TILE = 4096


def _body(z_ref, f_ref, embt_ref, wr_ref, wall_ref, b_ref, o_ref,
          wcomb_ref, bcomb_ref):
    @pl.when(pl.program_id(0) == 0)
    def _():
        embt = embt_ref[...]
        u = embt * (1.0 / (1.0 + jnp.exp(-embt)))
        logits = jnp.dot(wr_ref[...], u,
                         preferred_element_type=jnp.float32)
        row = lax.broadcasted_iota(jnp.int32, (N_EXP, N_SP_PAD), 0)
        valid = row < N_RTD
        lm = jnp.max(jnp.where(valid, logits, jnp.float32(-1e30)),
                     axis=0, keepdims=True)
        ex = jnp.where(valid, jnp.exp(logits - lm), 0.0)
        s = ex / jnp.sum(ex, axis=0, keepdims=True)
        m1 = jnp.max(s, axis=0, keepdims=True)
        i1 = jnp.min(jnp.where((s == m1) & valid, row, N_EXP),
                     axis=0, keepdims=True)
        msk2 = valid & (row != i1)
        sm = jnp.where(msk2, s, -1.0)
        m2 = jnp.max(sm, axis=0, keepdims=True)
        i2 = jnp.min(jnp.where(sm == m2, row, N_EXP), axis=0, keepdims=True)
        keep = (row == i1) | (row == i2)
        coef_t = jnp.where(valid, jnp.where(keep, s, 0.0), 1.0)
        wcomb_ref[...] = jnp.dot(wall_ref[...], coef_t,
                                 preferred_element_type=jnp.float32)
        bcomb_ref[...] = jnp.dot(b_ref[...], coef_t,
                                 preferred_element_type=jnp.float32)

    o = jnp.dot(f_ref[...], wcomb_ref[...],
                preferred_element_type=jnp.float32)
    p = o + bcomb_ref[0:1, :]
    z = z_ref[...]
    sp = lax.broadcasted_iota(jnp.int32, (z.shape[0], N_SP_PAD), 1)
    sel = (z == sp).astype(jnp.float32)
    o_ref[...] = jnp.sum(sel * p, axis=1, keepdims=True)


def kernel(features, species_idx, emb, W_router, W_experts, b_experts):
    n, in_f = features.shape
    n_species, embd = emb.shape
    wall = W_experts[:, 0, :].T
    wr = jnp.zeros((N_EXP, embd), jnp.float32).at[:N_RTD].set(W_router)
    embt = jnp.zeros((embd, N_SP_PAD), jnp.float32).at[:, :n_species].set(emb.T)
    b_rep = jnp.broadcast_to(b_experts.reshape(1, N_EXP), (8, N_EXP))
    z2d = species_idx.astype(jnp.int32).reshape(n, 1)

    out = pl.pallas_call(
        _body,
        grid=(n // TILE,),
        in_specs=[
            pl.BlockSpec((TILE, 1), lambda i: (i, 0)),
            pl.BlockSpec((TILE, in_f), lambda i: (i, 0)),
            pl.BlockSpec((embd, N_SP_PAD), lambda i: (0, 0)),
            pl.BlockSpec((N_EXP, embd), lambda i: (0, 0)),
            pl.BlockSpec((in_f, N_EXP), lambda i: (0, 0)),
            pl.BlockSpec((8, N_EXP), lambda i: (0, 0)),
        ],
        out_specs=pl.BlockSpec((TILE, 1), lambda i: (i, 0)),
        out_shape=jax.ShapeDtypeStruct((n, 1), jnp.float32),
        scratch_shapes=[
            pltpu.VMEM((in_f, N_SP_PAD), jnp.float32),
            pltpu.VMEM((8, N_SP_PAD), jnp.float32),
        ],
    )(z2d, features, embt, wr, wall, b_rep)
    return out
